# baseline (speedup 1.0000x reference)
import jax
import jax.numpy as jnp
from jax import lax
from jax.experimental import pallas as pl
from jax.experimental.pallas import tpu as pltpu
from jax.experimental.pallas import tpu_sc as plsc

NN = 100000
NE = 3200000
NG = 64
NPAD = 100096
NROWS = NPAD // 128
SUB = 16
CORES = 2
NW = CORES * SUB
PER_SUB = NPAD // SUB

RPW = 784
EP_ROWS = RPW * NW
EP = EP_ROWS * 128
U = 16
T_OUT = RPW // U

_mesh = plsc.VectorSubcoreMesh(core_axis_name="c", subcore_axis_name="s")


BE = U * 128
VUNR = 8


def _deg_body(dst_hbm, zeros_hbm, out_hbm, deg_pv, idx_v):
    c = lax.axis_index("c")
    s = lax.axis_index("s")
    wid = c * SUB + s
    pltpu.sync_copy(zeros_hbm, deg_pv)
    base_e = wid * RPW * 128
    ones16 = jnp.full((16,), 1.0, jnp.float32)

    def outer(i, carry):
        e0 = base_e + i * BE
        pltpu.sync_copy(dst_hbm.at[pl.ds(e0, BE)], idx_v)

        def inner(k, c2):
            for u in range(VUNR):
                iv = idx_v[pl.ds((k * VUNR + u) * 16, 16)]
                plsc.addupdate_scatter(deg_pv, [iv], ones16)
            return c2

        lax.fori_loop(0, BE // 16 // VUNR, inner, 0)
        return carry

    lax.fori_loop(0, T_OUT, outer, 0)
    pltpu.sync_copy(deg_pv, out_hbm.at[pl.ds(wid * NPAD, NPAD)])


_deg_call = pl.kernel(
    _deg_body,
    out_type=jax.ShapeDtypeStruct((NW * NPAD,), jnp.float32),
    mesh=_mesh,
    scratch_types=[
        pltpu.VMEM((NPAD,), jnp.float32),
        pltpu.VMEM((BE,), jnp.int32),
    ],
    compiler_params=pltpu.CompilerParams(needs_layout_passes=False),
)


def _p1_body(src_hbm, dst_hbm, v1_hbm, zeros_hbm, out_hbm,
             v1_sh, acc_pv, idx_s, idx_d, vals, stage_v):
    c = lax.axis_index("c")
    s = lax.axis_index("s")
    wid = c * SUB + s
    sl = pl.ds(s * PER_SUB, PER_SUB)
    pltpu.sync_copy(v1_hbm.at[sl], stage_v)
    pltpu.sync_copy(stage_v, v1_sh.at[sl])
    pltpu.sync_copy(zeros_hbm, acc_pv)
    plsc.subcore_barrier()
    base_e = wid * RPW * 128

    def outer(i, carry):
        e0 = base_e + i * BE
        pltpu.sync_copy(src_hbm.at[pl.ds(e0, BE)], idx_s)
        pltpu.sync_copy(dst_hbm.at[pl.ds(e0, BE)], idx_d)
        pltpu.sync_copy(v1_sh.at[idx_s], vals)

        def inner(k, c2):
            for u in range(VUNR):
                o = (k * VUNR + u) * 16
                iv = idx_d[pl.ds(o, 16)]
                vv = vals[pl.ds(o, 16)]
                plsc.addupdate_scatter(acc_pv, [iv], vv)
            return c2

        lax.fori_loop(0, BE // 16 // VUNR, inner, 0)
        return carry

    lax.fori_loop(0, T_OUT, outer, 0)
    pltpu.sync_copy(acc_pv, out_hbm.at[pl.ds(wid * NPAD, NPAD)])


_p1_call = pl.kernel(
    _p1_body,
    out_type=jax.ShapeDtypeStruct((NW * NPAD,), jnp.float32),
    mesh=_mesh,
    scratch_types=[
        pltpu.VMEM_SHARED((NPAD,), jnp.float32),
        pltpu.VMEM((NPAD,), jnp.float32),
        pltpu.VMEM((BE,), jnp.int32),
        pltpu.VMEM((BE,), jnp.int32),
        pltpu.VMEM((BE,), jnp.float32),
        pltpu.VMEM((PER_SUB,), jnp.float32),
    ],
    compiler_params=pltpu.CompilerParams(needs_layout_passes=False),
)


U2 = 16
T_OUT2 = RPW // U2


def _p2_body(src_hbm, dst_hbm, v2c0_hbm, v2c1_hbm, zeros_hbm, out0_hbm, out1_hbm,
             v0_sh, v1_sh, a0_sh, a1_sh, idx_s, idx_d, vals0, vals1, stage_v):
    c = lax.axis_index("c")
    s = lax.axis_index("s")
    wid = c * SUB + s
    sl = pl.ds(s * PER_SUB, PER_SUB)
    pltpu.sync_copy(v2c0_hbm.at[sl], stage_v)
    pltpu.sync_copy(stage_v, v0_sh.at[sl])
    pltpu.sync_copy(v2c1_hbm.at[sl], stage_v)
    pltpu.sync_copy(stage_v, v1_sh.at[sl])
    pltpu.sync_copy(zeros_hbm.at[sl], stage_v)
    pltpu.sync_copy(stage_v, a0_sh.at[sl])
    pltpu.sync_copy(stage_v, a1_sh.at[sl])
    plsc.subcore_barrier()
    base = wid * RPW

    def outer(i, carry):
        e0 = base * 128 + i * (U2 * 128)
        pltpu.sync_copy(src_hbm.at[pl.ds(e0, U2 * 128)], idx_s)
        pltpu.sync_copy(dst_hbm.at[pl.ds(e0, U2 * 128)], idx_d)
        pltpu.sync_copy(v0_sh.at[idx_s], vals0)
        pltpu.sync_copy(v1_sh.at[idx_s], vals1)
        pltpu.sync_copy(vals0, a0_sh.at[idx_d], add=True)
        pltpu.sync_copy(vals1, a1_sh.at[idx_d], add=True)
        return carry

    lax.fori_loop(0, T_OUT2, outer, 0)
    plsc.subcore_barrier()
    pltpu.sync_copy(a0_sh.at[sl], stage_v)
    pltpu.sync_copy(stage_v, out0_hbm.at[pl.ds(c * NPAD + s * PER_SUB, PER_SUB)])
    pltpu.sync_copy(a1_sh.at[sl], stage_v)
    pltpu.sync_copy(stage_v, out1_hbm.at[pl.ds(c * NPAD + s * PER_SUB, PER_SUB)])


_p2_call = pl.kernel(
    _p2_body,
    out_type=(
        jax.ShapeDtypeStruct((CORES * NPAD,), jnp.float32),
        jax.ShapeDtypeStruct((CORES * NPAD,), jnp.float32),
    ),
    mesh=_mesh,
    scratch_types=[
        pltpu.VMEM_SHARED((NPAD,), jnp.float32),
        pltpu.VMEM_SHARED((NPAD,), jnp.float32),
        pltpu.VMEM_SHARED((NPAD,), jnp.float32),
        pltpu.VMEM_SHARED((NPAD,), jnp.float32),
        pltpu.VMEM((U2 * 128,), jnp.int32),
        pltpu.VMEM((U2 * 128,), jnp.int32),
        pltpu.VMEM((U2 * 128,), jnp.float32),
        pltpu.VMEM((U2 * 128,), jnp.float32),
        pltpu.VMEM((PER_SUB,), jnp.float32),
    ],
)


def _prep_body(degp_ref, xp_ref, dinv_ref, v1_ref):
    deg = jnp.sum(degp_ref[...], axis=0) + 1.0
    dinv = lax.rsqrt(deg)
    dinv_ref[...] = dinv
    v1_ref[...] = xp_ref[...] * dinv


_prep_call = pl.pallas_call(
    _prep_body,
    out_shape=(
        jax.ShapeDtypeStruct((NROWS, 128), jnp.float32),
        jax.ShapeDtypeStruct((NROWS, 128), jnp.float32),
    ),
)


def _mid_body(accp_ref, dinv_ref, v1_ref, w1_ref, b1_ref, w2_ref, v2cm_ref):
    dinv = dinv_ref[...]
    s1 = dinv * (jnp.sum(accp_ref[...], axis=0) + v1_ref[...])
    g0 = jnp.zeros_like(s1)
    g1 = jnp.zeros_like(s1)
    for k in range(16):
        h = jnp.maximum(s1 * w1_ref[0, k] + b1_ref[0, k], 0.0)
        g0 += h * w2_ref[k, 0]
        g1 += h * w2_ref[k, 1]
    v2cm_ref[0] = g0 * dinv
    v2cm_ref[1] = g1 * dinv


_mid_call = pl.pallas_call(
    _mid_body,
    in_specs=[
        pl.BlockSpec(memory_space=pltpu.VMEM),
        pl.BlockSpec(memory_space=pltpu.VMEM),
        pl.BlockSpec(memory_space=pltpu.VMEM),
        pl.BlockSpec(memory_space=pltpu.SMEM),
        pl.BlockSpec(memory_space=pltpu.SMEM),
        pl.BlockSpec(memory_space=pltpu.SMEM),
    ],
    out_shape=jax.ShapeDtypeStruct((2, NROWS, 128), jnp.float32),
)


def _final_body(acc2cm_ref, v2cm_ref, dinv_ref, b2_ref, batch_ref, out_ref):
    dinv = dinv_ref[...]
    hs = []
    for ch in range(2):
        s2 = dinv * (acc2cm_ref[0, ch] + acc2cm_ref[1, ch] + v2cm_ref[ch])
        hs.append(jnp.maximum(s2 + b2_ref[0, ch], 0.0))
    bt = batch_ref[...]
    for g in range(NG):
        m = (bt == g).astype(jnp.float32)
        cnt = jnp.maximum(jnp.sum(m), 1.0)
        z0 = jnp.sum(m * hs[0]) / cnt
        z1 = jnp.sum(m * hs[1]) / cnt
        mx = jnp.maximum(z0, z1)
        lse = jnp.log(jnp.exp(z0 - mx) + jnp.exp(z1 - mx)) + mx
        out_ref[g, 0] = z0 - lse
        out_ref[g, 1] = z1 - lse


_final_call = pl.pallas_call(
    _final_body,
    in_specs=[
        pl.BlockSpec(memory_space=pltpu.VMEM),
        pl.BlockSpec(memory_space=pltpu.VMEM),
        pl.BlockSpec(memory_space=pltpu.VMEM),
        pl.BlockSpec(memory_space=pltpu.SMEM),
        pl.BlockSpec(memory_space=pltpu.VMEM),
    ],
    out_specs=pl.BlockSpec(memory_space=pltpu.SMEM),
    out_shape=jax.ShapeDtypeStruct((NG, 2), jnp.float32),
)


def kernel(x, edge_index, batch, W1, b1, W2, b2):
    ei = edge_index.astype(jnp.int32)
    npad_e = EP - NE
    pad_idx = NN + (jnp.arange(npad_e, dtype=jnp.int32) % (NPAD - NN))
    srcp = jnp.concatenate([ei[0], pad_idx])
    dstp = jnp.concatenate([ei[1], pad_idx])
    zerosN = jnp.zeros((NPAD,), jnp.float32)

    degp = _deg_call(dstp, zerosN)

    xp = jnp.concatenate([x[:, 0], jnp.zeros((NPAD - NN,), jnp.float32)])
    dinv, v1 = _prep_call(degp.reshape(NW, NROWS, 128), xp.reshape(NROWS, 128))

    acc1p = _p1_call(srcp, dstp, v1.reshape(NPAD), zerosN)

    v2cm = _mid_call(acc1p.reshape(NW, NROWS, 128), dinv, v1,
                     W1, b1.reshape(1, 16), W2)

    v2flat = v2cm.reshape(2, NPAD)
    acc2c0, acc2c1 = _p2_call(srcp, dstp, v2flat[0], v2flat[1], zerosN)

    acc2cm = jnp.stack(
        [acc2c0.reshape(CORES, NROWS, 128), acc2c1.reshape(CORES, NROWS, 128)],
        axis=1)
    batchp = jnp.concatenate(
        [batch.astype(jnp.int32), jnp.full((NPAD - NN,), NG, jnp.int32)]
    ).reshape(NROWS, 128)

    return _final_call(acc2cm, v2cm, dinv, b2.reshape(1, 2), batchp)


# speedup vs baseline: 1.1983x; 1.1983x over previous
import jax
import jax.numpy as jnp
from jax import lax
from jax.experimental import pallas as pl
from jax.experimental.pallas import tpu as pltpu
from jax.experimental.pallas import tpu_sc as plsc

NN = 100000
NE = 3200000
NG = 64
NPAD = 100096
NROWS = NPAD // 128
SUB = 16
CORES = 2
NW = CORES * SUB
PER_SUB = NPAD // SUB

RPW = 784
EP = RPW * 128 * NW
BE = 2048
EPW = RPW * 128
TB = EPW // BE
EP_ALLOC = EP + BE

_mesh = plsc.VectorSubcoreMesh(core_axis_name="c", subcore_axis_name="s")


def _start_idx(edge_hbm, e0, buf, sem):
    return pltpu.async_copy(edge_hbm.at[pl.ds(e0, BE)], buf, sem)


def _drain_idx(edge_hbm, buf, sem):
    pltpu.make_async_copy(edge_hbm.at[pl.ds(0, BE)], buf, sem).wait()


def _deg_body(dst_hbm, ones_hbm, zeros_hbm, out_hbm,
              deg_sh, idx_a, idx_b, ones_v, stage_v, sem_a, sem_b):
    c = lax.axis_index("c")
    s = lax.axis_index("s")
    wid = c * SUB + s
    sl = pl.ds(s * PER_SUB, PER_SUB)
    base_e = wid * EPW
    _start_idx(dst_hbm, base_e, idx_a, sem_a)
    pltpu.sync_copy(zeros_hbm.at[sl], stage_v)
    pltpu.sync_copy(stage_v, deg_sh.at[sl])
    pltpu.sync_copy(ones_hbm, ones_v)
    plsc.subcore_barrier()

    def outer(i, carry):
        e0 = base_e + i * (2 * BE)
        _start_idx(dst_hbm, e0 + BE, idx_b, sem_b)
        _drain_idx(dst_hbm, idx_a, sem_a)
        pltpu.sync_copy(ones_v, deg_sh.at[idx_a], add=True)
        _start_idx(dst_hbm, e0 + 2 * BE, idx_a, sem_a)
        _drain_idx(dst_hbm, idx_b, sem_b)
        pltpu.sync_copy(ones_v, deg_sh.at[idx_b], add=True)
        return carry

    lax.fori_loop(0, TB // 2, outer, 0)
    _drain_idx(dst_hbm, idx_a, sem_a)
    pltpu.sync_copy(ones_v, deg_sh.at[idx_a], add=True)
    plsc.subcore_barrier()
    pltpu.sync_copy(deg_sh.at[sl], stage_v)
    pltpu.sync_copy(stage_v, out_hbm.at[pl.ds(c * NPAD + s * PER_SUB, PER_SUB)])


_deg_call = pl.kernel(
    _deg_body,
    out_type=jax.ShapeDtypeStruct((CORES * NPAD,), jnp.float32),
    mesh=_mesh,
    scratch_types=[
        pltpu.VMEM_SHARED((NPAD,), jnp.float32),
        pltpu.VMEM((BE,), jnp.int32),
        pltpu.VMEM((BE,), jnp.int32),
        pltpu.VMEM((BE,), jnp.float32),
        pltpu.VMEM((PER_SUB,), jnp.float32),
        pltpu.SemaphoreType.DMA,
        pltpu.SemaphoreType.DMA,
    ],
)


def _p1_body(src_hbm, dst_hbm, v1_hbm, zeros_hbm, out_hbm,
             v1_sh, acc_sh, idx_sa, idx_da, idx_sb, idx_db, vals, stage_v,
             sem_a, sem_b):
    c = lax.axis_index("c")
    s = lax.axis_index("s")
    wid = c * SUB + s
    sl = pl.ds(s * PER_SUB, PER_SUB)
    base_e = wid * EPW
    _start_idx(src_hbm, base_e, idx_sa, sem_a)
    _start_idx(dst_hbm, base_e, idx_da, sem_a)
    pltpu.sync_copy(v1_hbm.at[sl], stage_v)
    pltpu.sync_copy(stage_v, v1_sh.at[sl])
    pltpu.sync_copy(zeros_hbm.at[sl], stage_v)
    pltpu.sync_copy(stage_v, acc_sh.at[sl])
    plsc.subcore_barrier()

    def outer(i, carry):
        e0 = base_e + i * (2 * BE)
        _start_idx(src_hbm, e0 + BE, idx_sb, sem_b)
        _start_idx(dst_hbm, e0 + BE, idx_db, sem_b)
        _drain_idx(src_hbm, idx_sa, sem_a)
        _drain_idx(dst_hbm, idx_da, sem_a)
        pltpu.sync_copy(v1_sh.at[idx_sa], vals)
        pltpu.sync_copy(vals, acc_sh.at[idx_da], add=True)
        _start_idx(src_hbm, e0 + 2 * BE, idx_sa, sem_a)
        _start_idx(dst_hbm, e0 + 2 * BE, idx_da, sem_a)
        _drain_idx(src_hbm, idx_sb, sem_b)
        _drain_idx(dst_hbm, idx_db, sem_b)
        pltpu.sync_copy(v1_sh.at[idx_sb], vals)
        pltpu.sync_copy(vals, acc_sh.at[idx_db], add=True)
        return carry

    lax.fori_loop(0, TB // 2, outer, 0)
    _drain_idx(src_hbm, idx_sa, sem_a)
    _drain_idx(dst_hbm, idx_da, sem_a)
    pltpu.sync_copy(v1_sh.at[idx_sa], vals)
    pltpu.sync_copy(vals, acc_sh.at[idx_da], add=True)
    plsc.subcore_barrier()
    pltpu.sync_copy(acc_sh.at[sl], stage_v)
    pltpu.sync_copy(stage_v, out_hbm.at[pl.ds(c * NPAD + s * PER_SUB, PER_SUB)])


_p1_call = pl.kernel(
    _p1_body,
    out_type=jax.ShapeDtypeStruct((CORES * NPAD,), jnp.float32),
    mesh=_mesh,
    scratch_types=[
        pltpu.VMEM_SHARED((NPAD,), jnp.float32),
        pltpu.VMEM_SHARED((NPAD,), jnp.float32),
        pltpu.VMEM((BE,), jnp.int32),
        pltpu.VMEM((BE,), jnp.int32),
        pltpu.VMEM((BE,), jnp.int32),
        pltpu.VMEM((BE,), jnp.int32),
        pltpu.VMEM((BE,), jnp.float32),
        pltpu.VMEM((PER_SUB,), jnp.float32),
        pltpu.SemaphoreType.DMA,
        pltpu.SemaphoreType.DMA,
    ],
)


def _p2_body(src_hbm, dst_hbm, v2c0_hbm, v2c1_hbm, zeros_hbm, out0_hbm, out1_hbm,
             v0_sh, w1_sh, a0_sh, a1_sh, idx_sa, idx_da, idx_sb, idx_db,
             vals0, vals1, stage_v, sem_a, sem_b):
    c = lax.axis_index("c")
    s = lax.axis_index("s")
    wid = c * SUB + s
    sl = pl.ds(s * PER_SUB, PER_SUB)
    base_e = wid * EPW
    _start_idx(src_hbm, base_e, idx_sa, sem_a)
    _start_idx(dst_hbm, base_e, idx_da, sem_a)
    pltpu.sync_copy(v2c0_hbm.at[sl], stage_v)
    pltpu.sync_copy(stage_v, v0_sh.at[sl])
    pltpu.sync_copy(v2c1_hbm.at[sl], stage_v)
    pltpu.sync_copy(stage_v, w1_sh.at[sl])
    pltpu.sync_copy(zeros_hbm.at[sl], stage_v)
    pltpu.sync_copy(stage_v, a0_sh.at[sl])
    pltpu.sync_copy(stage_v, a1_sh.at[sl])
    plsc.subcore_barrier()

    def do_batch(idx_s, idx_d):
        pltpu.sync_copy(v0_sh.at[idx_s], vals0)
        pltpu.sync_copy(w1_sh.at[idx_s], vals1)
        pltpu.sync_copy(vals0, a0_sh.at[idx_d], add=True)
        pltpu.sync_copy(vals1, a1_sh.at[idx_d], add=True)

    def outer(i, carry):
        e0 = base_e + i * (2 * BE)
        _start_idx(src_hbm, e0 + BE, idx_sb, sem_b)
        _start_idx(dst_hbm, e0 + BE, idx_db, sem_b)
        _drain_idx(src_hbm, idx_sa, sem_a)
        _drain_idx(dst_hbm, idx_da, sem_a)
        do_batch(idx_sa, idx_da)
        _start_idx(src_hbm, e0 + 2 * BE, idx_sa, sem_a)
        _start_idx(dst_hbm, e0 + 2 * BE, idx_da, sem_a)
        _drain_idx(src_hbm, idx_sb, sem_b)
        _drain_idx(dst_hbm, idx_db, sem_b)
        do_batch(idx_sb, idx_db)
        return carry

    lax.fori_loop(0, TB // 2, outer, 0)
    _drain_idx(src_hbm, idx_sa, sem_a)
    _drain_idx(dst_hbm, idx_da, sem_a)
    do_batch(idx_sa, idx_da)
    plsc.subcore_barrier()
    pltpu.sync_copy(a0_sh.at[sl], stage_v)
    pltpu.sync_copy(stage_v, out0_hbm.at[pl.ds(c * NPAD + s * PER_SUB, PER_SUB)])
    pltpu.sync_copy(a1_sh.at[sl], stage_v)
    pltpu.sync_copy(stage_v, out1_hbm.at[pl.ds(c * NPAD + s * PER_SUB, PER_SUB)])


_p2_call = pl.kernel(
    _p2_body,
    out_type=(
        jax.ShapeDtypeStruct((CORES * NPAD,), jnp.float32),
        jax.ShapeDtypeStruct((CORES * NPAD,), jnp.float32),
    ),
    mesh=_mesh,
    scratch_types=[
        pltpu.VMEM_SHARED((NPAD,), jnp.float32),
        pltpu.VMEM_SHARED((NPAD,), jnp.float32),
        pltpu.VMEM_SHARED((NPAD,), jnp.float32),
        pltpu.VMEM_SHARED((NPAD,), jnp.float32),
        pltpu.VMEM((BE,), jnp.int32),
        pltpu.VMEM((BE,), jnp.int32),
        pltpu.VMEM((BE,), jnp.int32),
        pltpu.VMEM((BE,), jnp.int32),
        pltpu.VMEM((BE,), jnp.float32),
        pltpu.VMEM((BE,), jnp.float32),
        pltpu.VMEM((PER_SUB,), jnp.float32),
        pltpu.SemaphoreType.DMA,
        pltpu.SemaphoreType.DMA,
    ],
)


def _prep_body(degp_ref, xp_ref, dinv_ref, v1_ref):
    deg = degp_ref[0] + degp_ref[1] + 1.0
    dinv = lax.rsqrt(deg)
    dinv_ref[...] = dinv
    v1_ref[...] = xp_ref[...] * dinv


_prep_call = pl.pallas_call(
    _prep_body,
    out_shape=(
        jax.ShapeDtypeStruct((NROWS, 128), jnp.float32),
        jax.ShapeDtypeStruct((NROWS, 128), jnp.float32),
    ),
)


def _mid_body(accp_ref, dinv_ref, v1_ref, w1_ref, b1_ref, w2_ref, v2cm_ref):
    dinv = dinv_ref[...]
    s1 = dinv * (accp_ref[0] + accp_ref[1] + v1_ref[...])
    g0 = jnp.zeros_like(s1)
    g1 = jnp.zeros_like(s1)
    for k in range(16):
        h = jnp.maximum(s1 * w1_ref[0, k] + b1_ref[0, k], 0.0)
        g0 += h * w2_ref[k, 0]
        g1 += h * w2_ref[k, 1]
    v2cm_ref[0] = g0 * dinv
    v2cm_ref[1] = g1 * dinv


_mid_call = pl.pallas_call(
    _mid_body,
    in_specs=[
        pl.BlockSpec(memory_space=pltpu.VMEM),
        pl.BlockSpec(memory_space=pltpu.VMEM),
        pl.BlockSpec(memory_space=pltpu.VMEM),
        pl.BlockSpec(memory_space=pltpu.SMEM),
        pl.BlockSpec(memory_space=pltpu.SMEM),
        pl.BlockSpec(memory_space=pltpu.SMEM),
    ],
    out_shape=jax.ShapeDtypeStruct((2, NROWS, 128), jnp.float32),
)


def _final_body(acc2cm_ref, v2cm_ref, dinv_ref, b2_ref, batch_ref, out_ref):
    dinv = dinv_ref[...]
    hs = []
    for ch in range(2):
        s2 = dinv * (acc2cm_ref[0, ch] + acc2cm_ref[1, ch] + v2cm_ref[ch])
        hs.append(jnp.maximum(s2 + b2_ref[0, ch], 0.0))
    bt = batch_ref[...]
    for g in range(NG):
        m = (bt == g).astype(jnp.float32)
        cnt = jnp.maximum(jnp.sum(m), 1.0)
        z0 = jnp.sum(m * hs[0]) / cnt
        z1 = jnp.sum(m * hs[1]) / cnt
        mx = jnp.maximum(z0, z1)
        lse = jnp.log(jnp.exp(z0 - mx) + jnp.exp(z1 - mx)) + mx
        out_ref[g, 0] = z0 - lse
        out_ref[g, 1] = z1 - lse


_final_call = pl.pallas_call(
    _final_body,
    in_specs=[
        pl.BlockSpec(memory_space=pltpu.VMEM),
        pl.BlockSpec(memory_space=pltpu.VMEM),
        pl.BlockSpec(memory_space=pltpu.VMEM),
        pl.BlockSpec(memory_space=pltpu.SMEM),
        pl.BlockSpec(memory_space=pltpu.VMEM),
    ],
    out_specs=pl.BlockSpec(memory_space=pltpu.SMEM),
    out_shape=jax.ShapeDtypeStruct((NG, 2), jnp.float32),
)


def kernel(x, edge_index, batch, W1, b1, W2, b2):
    ei = edge_index.astype(jnp.int32)
    npad_e = EP_ALLOC - NE
    pad_idx = NN + (jnp.arange(npad_e, dtype=jnp.int32) % (NPAD - NN))
    srcp = jnp.concatenate([ei[0], pad_idx])
    dstp = jnp.concatenate([ei[1], pad_idx])
    zerosN = jnp.zeros((NPAD,), jnp.float32)
    ones_b = jnp.ones((BE,), jnp.float32)

    degp = _deg_call(dstp, ones_b, zerosN)

    xp = jnp.concatenate([x[:, 0], jnp.zeros((NPAD - NN,), jnp.float32)])
    dinv, v1 = _prep_call(degp.reshape(CORES, NROWS, 128), xp.reshape(NROWS, 128))

    acc1p = _p1_call(srcp, dstp, v1.reshape(NPAD), zerosN)

    v2cm = _mid_call(acc1p.reshape(CORES, NROWS, 128), dinv, v1,
                     W1, b1.reshape(1, 16), W2)

    v2flat = v2cm.reshape(2, NPAD)
    acc2c0, acc2c1 = _p2_call(srcp, dstp, v2flat[0], v2flat[1], zerosN)

    acc2cm = jnp.stack(
        [acc2c0.reshape(CORES, NROWS, 128), acc2c1.reshape(CORES, NROWS, 128)],
        axis=1)
    batchp = jnp.concatenate(
        [batch.astype(jnp.int32), jnp.full((NPAD - NN,), NG, jnp.int32)]
    ).reshape(NROWS, 128)

    return _final_call(acc2cm, v2cm, dinv, b2.reshape(1, 2), batchp)


# speedup vs baseline: 1.6683x; 1.3922x over previous
import jax
import jax.numpy as jnp
from jax import lax
from jax.experimental import pallas as pl
from jax.experimental.pallas import tpu as pltpu
from jax.experimental.pallas import tpu_sc as plsc

NN = 100000
NE = 3200000
NG = 64
NPAD = 100096
NROWS = NPAD // 128
SUB = 16
CORES = 2
NW = CORES * SUB
PER_SUB = NPAD // SUB

RPW = 784
EP = RPW * 128 * NW
BE = 6272
EPW = RPW * 128
TB = EPW // BE
EP_ALLOC = EP + BE

_mesh = plsc.VectorSubcoreMesh(core_axis_name="c", subcore_axis_name="s")


def _start_idx(edge_hbm, e0, buf, sem):
    return pltpu.async_copy(edge_hbm.at[pl.ds(e0, BE)], buf, sem)


def _drain_idx(edge_hbm, buf, sem):
    pltpu.make_async_copy(edge_hbm.at[pl.ds(0, BE)], buf, sem).wait()


def _deg_body(dst_hbm, ones_hbm, zeros_hbm, out_hbm,
              deg_sh, idx_a, idx_b, ones_v, stage_v, sem_a, sem_b):
    c = lax.axis_index("c")
    s = lax.axis_index("s")
    wid = c * SUB + s
    sl = pl.ds(s * PER_SUB, PER_SUB)
    base_e = wid * EPW
    _start_idx(dst_hbm, base_e, idx_a, sem_a)
    pltpu.sync_copy(zeros_hbm.at[sl], stage_v)
    pltpu.sync_copy(stage_v, deg_sh.at[sl])
    pltpu.sync_copy(ones_hbm, ones_v)
    plsc.subcore_barrier()

    def outer(i, carry):
        e0 = base_e + i * (2 * BE)
        _start_idx(dst_hbm, e0 + BE, idx_b, sem_b)
        _drain_idx(dst_hbm, idx_a, sem_a)
        pltpu.sync_copy(ones_v, deg_sh.at[idx_a], add=True)
        _start_idx(dst_hbm, e0 + 2 * BE, idx_a, sem_a)
        _drain_idx(dst_hbm, idx_b, sem_b)
        pltpu.sync_copy(ones_v, deg_sh.at[idx_b], add=True)
        return carry

    lax.fori_loop(0, TB // 2, outer, 0)
    _drain_idx(dst_hbm, idx_a, sem_a)
    plsc.subcore_barrier()
    pltpu.sync_copy(deg_sh.at[sl], stage_v)
    pltpu.sync_copy(stage_v, out_hbm.at[pl.ds(c * NPAD + s * PER_SUB, PER_SUB)])


_deg_call = pl.kernel(
    _deg_body,
    out_type=jax.ShapeDtypeStruct((CORES * NPAD,), jnp.float32),
    mesh=_mesh,
    scratch_types=[
        pltpu.VMEM_SHARED((NPAD,), jnp.float32),
        pltpu.VMEM((BE,), jnp.int32),
        pltpu.VMEM((BE,), jnp.int32),
        pltpu.VMEM((BE,), jnp.float32),
        pltpu.VMEM((PER_SUB,), jnp.float32),
        pltpu.SemaphoreType.DMA,
        pltpu.SemaphoreType.DMA,
    ],
)


def _p1_body(src_hbm, dst_hbm, v1_hbm, zeros_hbm, out_hbm,
             v1_sh, acc_sh, idx_sa, idx_da, idx_sb, idx_db, vals, stage_v,
             sem_a, sem_b):
    c = lax.axis_index("c")
    s = lax.axis_index("s")
    wid = c * SUB + s
    sl = pl.ds(s * PER_SUB, PER_SUB)
    base_e = wid * EPW
    _start_idx(src_hbm, base_e, idx_sa, sem_a)
    _start_idx(dst_hbm, base_e, idx_da, sem_a)
    pltpu.sync_copy(v1_hbm.at[sl], stage_v)
    pltpu.sync_copy(stage_v, v1_sh.at[sl])
    pltpu.sync_copy(zeros_hbm.at[sl], stage_v)
    pltpu.sync_copy(stage_v, acc_sh.at[sl])
    plsc.subcore_barrier()

    def outer(i, carry):
        e0 = base_e + i * (2 * BE)
        _start_idx(src_hbm, e0 + BE, idx_sb, sem_b)
        _start_idx(dst_hbm, e0 + BE, idx_db, sem_b)
        _drain_idx(src_hbm, idx_sa, sem_a)
        _drain_idx(dst_hbm, idx_da, sem_a)
        pltpu.sync_copy(v1_sh.at[idx_sa], vals)
        pltpu.sync_copy(vals, acc_sh.at[idx_da], add=True)
        _start_idx(src_hbm, e0 + 2 * BE, idx_sa, sem_a)
        _start_idx(dst_hbm, e0 + 2 * BE, idx_da, sem_a)
        _drain_idx(src_hbm, idx_sb, sem_b)
        _drain_idx(dst_hbm, idx_db, sem_b)
        pltpu.sync_copy(v1_sh.at[idx_sb], vals)
        pltpu.sync_copy(vals, acc_sh.at[idx_db], add=True)
        return carry

    lax.fori_loop(0, TB // 2, outer, 0)
    _drain_idx(src_hbm, idx_sa, sem_a)
    _drain_idx(dst_hbm, idx_da, sem_a)
    plsc.subcore_barrier()
    pltpu.sync_copy(acc_sh.at[sl], stage_v)
    pltpu.sync_copy(stage_v, out_hbm.at[pl.ds(c * NPAD + s * PER_SUB, PER_SUB)])


_p1_call = pl.kernel(
    _p1_body,
    out_type=jax.ShapeDtypeStruct((CORES * NPAD,), jnp.float32),
    mesh=_mesh,
    scratch_types=[
        pltpu.VMEM_SHARED((NPAD,), jnp.float32),
        pltpu.VMEM_SHARED((NPAD,), jnp.float32),
        pltpu.VMEM((BE,), jnp.int32),
        pltpu.VMEM((BE,), jnp.int32),
        pltpu.VMEM((BE,), jnp.int32),
        pltpu.VMEM((BE,), jnp.int32),
        pltpu.VMEM((BE,), jnp.float32),
        pltpu.VMEM((PER_SUB,), jnp.float32),
        pltpu.SemaphoreType.DMA,
        pltpu.SemaphoreType.DMA,
    ],
)


def _p2_body(src_hbm, dst_hbm, v2c0_hbm, v2c1_hbm, zeros_hbm, out0_hbm, out1_hbm,
             v0_sh, w1_sh, a0_sh, a1_sh, idx_sa, idx_da, idx_sb, idx_db,
             vals0, vals1, stage_v, sem_a, sem_b):
    c = lax.axis_index("c")
    s = lax.axis_index("s")
    wid = c * SUB + s
    sl = pl.ds(s * PER_SUB, PER_SUB)
    base_e = wid * EPW
    _start_idx(src_hbm, base_e, idx_sa, sem_a)
    _start_idx(dst_hbm, base_e, idx_da, sem_a)
    pltpu.sync_copy(v2c0_hbm.at[sl], stage_v)
    pltpu.sync_copy(stage_v, v0_sh.at[sl])
    pltpu.sync_copy(v2c1_hbm.at[sl], stage_v)
    pltpu.sync_copy(stage_v, w1_sh.at[sl])
    pltpu.sync_copy(zeros_hbm.at[sl], stage_v)
    pltpu.sync_copy(stage_v, a0_sh.at[sl])
    pltpu.sync_copy(stage_v, a1_sh.at[sl])
    plsc.subcore_barrier()

    def do_batch(idx_s, idx_d):
        pltpu.sync_copy(v0_sh.at[idx_s], vals0)
        pltpu.sync_copy(w1_sh.at[idx_s], vals1)
        pltpu.sync_copy(vals0, a0_sh.at[idx_d], add=True)
        pltpu.sync_copy(vals1, a1_sh.at[idx_d], add=True)

    def outer(i, carry):
        e0 = base_e + i * (2 * BE)
        _start_idx(src_hbm, e0 + BE, idx_sb, sem_b)
        _start_idx(dst_hbm, e0 + BE, idx_db, sem_b)
        _drain_idx(src_hbm, idx_sa, sem_a)
        _drain_idx(dst_hbm, idx_da, sem_a)
        do_batch(idx_sa, idx_da)
        _start_idx(src_hbm, e0 + 2 * BE, idx_sa, sem_a)
        _start_idx(dst_hbm, e0 + 2 * BE, idx_da, sem_a)
        _drain_idx(src_hbm, idx_sb, sem_b)
        _drain_idx(dst_hbm, idx_db, sem_b)
        do_batch(idx_sb, idx_db)
        return carry

    lax.fori_loop(0, TB // 2, outer, 0)
    _drain_idx(src_hbm, idx_sa, sem_a)
    _drain_idx(dst_hbm, idx_da, sem_a)
    plsc.subcore_barrier()
    pltpu.sync_copy(a0_sh.at[sl], stage_v)
    pltpu.sync_copy(stage_v, out0_hbm.at[pl.ds(c * NPAD + s * PER_SUB, PER_SUB)])
    pltpu.sync_copy(a1_sh.at[sl], stage_v)
    pltpu.sync_copy(stage_v, out1_hbm.at[pl.ds(c * NPAD + s * PER_SUB, PER_SUB)])


_p2_call = pl.kernel(
    _p2_body,
    out_type=(
        jax.ShapeDtypeStruct((CORES * NPAD,), jnp.float32),
        jax.ShapeDtypeStruct((CORES * NPAD,), jnp.float32),
    ),
    mesh=_mesh,
    scratch_types=[
        pltpu.VMEM_SHARED((NPAD,), jnp.float32),
        pltpu.VMEM_SHARED((NPAD,), jnp.float32),
        pltpu.VMEM_SHARED((NPAD,), jnp.float32),
        pltpu.VMEM_SHARED((NPAD,), jnp.float32),
        pltpu.VMEM((BE,), jnp.int32),
        pltpu.VMEM((BE,), jnp.int32),
        pltpu.VMEM((BE,), jnp.int32),
        pltpu.VMEM((BE,), jnp.int32),
        pltpu.VMEM((BE,), jnp.float32),
        pltpu.VMEM((BE,), jnp.float32),
        pltpu.VMEM((PER_SUB,), jnp.float32),
        pltpu.SemaphoreType.DMA,
        pltpu.SemaphoreType.DMA,
    ],
)


def _prep_body(degp_ref, xp_ref, dinv_ref, v1_ref):
    deg = degp_ref[0] + degp_ref[1] + 1.0
    dinv = lax.rsqrt(deg)
    dinv_ref[...] = dinv
    v1_ref[...] = xp_ref[...] * dinv


_prep_call = pl.pallas_call(
    _prep_body,
    out_shape=(
        jax.ShapeDtypeStruct((NROWS, 128), jnp.float32),
        jax.ShapeDtypeStruct((NROWS, 128), jnp.float32),
    ),
)


def _mid_body(accp_ref, dinv_ref, v1_ref, w1_ref, b1_ref, w2_ref, v2cm_ref):
    dinv = dinv_ref[...]
    s1 = dinv * (accp_ref[0] + accp_ref[1] + v1_ref[...])
    g0 = jnp.zeros_like(s1)
    g1 = jnp.zeros_like(s1)
    for k in range(16):
        h = jnp.maximum(s1 * w1_ref[0, k] + b1_ref[0, k], 0.0)
        g0 += h * w2_ref[k, 0]
        g1 += h * w2_ref[k, 1]
    v2cm_ref[0] = g0 * dinv
    v2cm_ref[1] = g1 * dinv


_mid_call = pl.pallas_call(
    _mid_body,
    in_specs=[
        pl.BlockSpec(memory_space=pltpu.VMEM),
        pl.BlockSpec(memory_space=pltpu.VMEM),
        pl.BlockSpec(memory_space=pltpu.VMEM),
        pl.BlockSpec(memory_space=pltpu.SMEM),
        pl.BlockSpec(memory_space=pltpu.SMEM),
        pl.BlockSpec(memory_space=pltpu.SMEM),
    ],
    out_shape=jax.ShapeDtypeStruct((2, NROWS, 128), jnp.float32),
)


def _final_body(acc2cm_ref, v2cm_ref, dinv_ref, b2_ref, batch_ref, out_ref):
    dinv = dinv_ref[...]
    hs = []
    for ch in range(2):
        s2 = dinv * (acc2cm_ref[0, ch] + acc2cm_ref[1, ch] + v2cm_ref[ch])
        hs.append(jnp.maximum(s2 + b2_ref[0, ch], 0.0))
    bt = batch_ref[...]
    for g in range(NG):
        m = (bt == g).astype(jnp.float32)
        cnt = jnp.maximum(jnp.sum(m), 1.0)
        z0 = jnp.sum(m * hs[0]) / cnt
        z1 = jnp.sum(m * hs[1]) / cnt
        mx = jnp.maximum(z0, z1)
        lse = jnp.log(jnp.exp(z0 - mx) + jnp.exp(z1 - mx)) + mx
        out_ref[g, 0] = z0 - lse
        out_ref[g, 1] = z1 - lse


_final_call = pl.pallas_call(
    _final_body,
    in_specs=[
        pl.BlockSpec(memory_space=pltpu.VMEM),
        pl.BlockSpec(memory_space=pltpu.VMEM),
        pl.BlockSpec(memory_space=pltpu.VMEM),
        pl.BlockSpec(memory_space=pltpu.SMEM),
        pl.BlockSpec(memory_space=pltpu.VMEM),
    ],
    out_specs=pl.BlockSpec(memory_space=pltpu.SMEM),
    out_shape=jax.ShapeDtypeStruct((NG, 2), jnp.float32),
)


def kernel(x, edge_index, batch, W1, b1, W2, b2):
    ei = edge_index.astype(jnp.int32)
    npad_e = EP_ALLOC - NE
    pad_idx = NN + (jnp.arange(npad_e, dtype=jnp.int32) % (NPAD - NN))
    srcp = jnp.concatenate([ei[0], pad_idx])
    dstp = jnp.concatenate([ei[1], pad_idx])
    zerosN = jnp.zeros((NPAD,), jnp.float32)
    ones_b = jnp.ones((BE,), jnp.float32)

    degp = _deg_call(dstp, ones_b, zerosN)

    xp = jnp.concatenate([x[:, 0], jnp.zeros((NPAD - NN,), jnp.float32)])
    dinv, v1 = _prep_call(degp.reshape(CORES, NROWS, 128), xp.reshape(NROWS, 128))

    acc1p = _p1_call(srcp, dstp, v1.reshape(NPAD), zerosN)

    v2cm = _mid_call(acc1p.reshape(CORES, NROWS, 128), dinv, v1,
                     W1, b1.reshape(1, 16), W2)

    v2flat = v2cm.reshape(2, NPAD)
    acc2c0, acc2c1 = _p2_call(srcp, dstp, v2flat[0], v2flat[1], zerosN)

    acc2cm = jnp.stack(
        [acc2c0.reshape(CORES, NROWS, 128), acc2c1.reshape(CORES, NROWS, 128)],
        axis=1)
    batchp = jnp.concatenate(
        [batch.astype(jnp.int32), jnp.full((NPAD - NN,), NG, jnp.int32)]
    ).reshape(NROWS, 128)

    return _final_call(acc2cm, v2cm, dinv, b2.reshape(1, 2), batchp)


# speedup vs baseline: 1.7424x; 1.0444x over previous
import jax
import jax.numpy as jnp
from jax import lax
from jax.experimental import pallas as pl
from jax.experimental.pallas import tpu as pltpu
from jax.experimental.pallas import tpu_sc as plsc

NN = 100000
NE = 3200000
NG = 64
NPAD = 100096
NROWS = NPAD // 128
SUB = 16
CORES = 2
NW = CORES * SUB
PER_SUB = NPAD // SUB

RPW = 784
EP = RPW * 128 * NW
BE = 12544
EPW = RPW * 128
TB = EPW // BE
EP_ALLOC = EP + BE

_mesh = plsc.VectorSubcoreMesh(core_axis_name="c", subcore_axis_name="s")


def _start_idx(edge_hbm, e0, buf, sem):
    return pltpu.async_copy(edge_hbm.at[pl.ds(e0, BE)], buf, sem)


def _drain_idx(edge_hbm, buf, sem):
    pltpu.make_async_copy(edge_hbm.at[pl.ds(0, BE)], buf, sem).wait()


def _deg_body(dst_hbm, ones_hbm, zeros_hbm, out_hbm,
              deg_sh, idx_a, idx_b, ones_v, stage_v, sem_a, sem_b):
    c = lax.axis_index("c")
    s = lax.axis_index("s")
    wid = c * SUB + s
    sl = pl.ds(s * PER_SUB, PER_SUB)
    base_e = wid * EPW
    _start_idx(dst_hbm, base_e, idx_a, sem_a)
    pltpu.sync_copy(zeros_hbm.at[sl], stage_v)
    pltpu.sync_copy(stage_v, deg_sh.at[sl])
    pltpu.sync_copy(ones_hbm, ones_v)
    plsc.subcore_barrier()

    def outer(i, carry):
        e0 = base_e + i * (2 * BE)
        _start_idx(dst_hbm, e0 + BE, idx_b, sem_b)
        _drain_idx(dst_hbm, idx_a, sem_a)
        pltpu.sync_copy(ones_v, deg_sh.at[idx_a], add=True)
        _start_idx(dst_hbm, e0 + 2 * BE, idx_a, sem_a)
        _drain_idx(dst_hbm, idx_b, sem_b)
        pltpu.sync_copy(ones_v, deg_sh.at[idx_b], add=True)
        return carry

    lax.fori_loop(0, TB // 2, outer, 0)
    _drain_idx(dst_hbm, idx_a, sem_a)
    plsc.subcore_barrier()
    pltpu.sync_copy(deg_sh.at[sl], stage_v)
    pltpu.sync_copy(stage_v, out_hbm.at[pl.ds(c * NPAD + s * PER_SUB, PER_SUB)])


_deg_call = pl.kernel(
    _deg_body,
    out_type=jax.ShapeDtypeStruct((CORES * NPAD,), jnp.float32),
    mesh=_mesh,
    scratch_types=[
        pltpu.VMEM_SHARED((NPAD,), jnp.float32),
        pltpu.VMEM((BE,), jnp.int32),
        pltpu.VMEM((BE,), jnp.int32),
        pltpu.VMEM((BE,), jnp.float32),
        pltpu.VMEM((PER_SUB,), jnp.float32),
        pltpu.SemaphoreType.DMA,
        pltpu.SemaphoreType.DMA,
    ],
)


def _p1_body(src_hbm, dst_hbm, v1_hbm, zeros_hbm, out_hbm,
             v1_sh, acc_sh, idx_sa, idx_da, idx_sb, idx_db, vals, stage_v,
             sem_a, sem_b):
    c = lax.axis_index("c")
    s = lax.axis_index("s")
    wid = c * SUB + s
    sl = pl.ds(s * PER_SUB, PER_SUB)
    base_e = wid * EPW
    _start_idx(src_hbm, base_e, idx_sa, sem_a)
    _start_idx(dst_hbm, base_e, idx_da, sem_a)
    pltpu.sync_copy(v1_hbm.at[sl], stage_v)
    pltpu.sync_copy(stage_v, v1_sh.at[sl])
    pltpu.sync_copy(zeros_hbm.at[sl], stage_v)
    pltpu.sync_copy(stage_v, acc_sh.at[sl])
    plsc.subcore_barrier()

    def outer(i, carry):
        e0 = base_e + i * (2 * BE)
        _start_idx(src_hbm, e0 + BE, idx_sb, sem_b)
        _start_idx(dst_hbm, e0 + BE, idx_db, sem_b)
        _drain_idx(src_hbm, idx_sa, sem_a)
        _drain_idx(dst_hbm, idx_da, sem_a)
        pltpu.sync_copy(v1_sh.at[idx_sa], vals)
        pltpu.sync_copy(vals, acc_sh.at[idx_da], add=True)
        _start_idx(src_hbm, e0 + 2 * BE, idx_sa, sem_a)
        _start_idx(dst_hbm, e0 + 2 * BE, idx_da, sem_a)
        _drain_idx(src_hbm, idx_sb, sem_b)
        _drain_idx(dst_hbm, idx_db, sem_b)
        pltpu.sync_copy(v1_sh.at[idx_sb], vals)
        pltpu.sync_copy(vals, acc_sh.at[idx_db], add=True)
        return carry

    lax.fori_loop(0, TB // 2, outer, 0)
    _drain_idx(src_hbm, idx_sa, sem_a)
    _drain_idx(dst_hbm, idx_da, sem_a)
    plsc.subcore_barrier()
    pltpu.sync_copy(acc_sh.at[sl], stage_v)
    pltpu.sync_copy(stage_v, out_hbm.at[pl.ds(c * NPAD + s * PER_SUB, PER_SUB)])


_p1_call = pl.kernel(
    _p1_body,
    out_type=jax.ShapeDtypeStruct((CORES * NPAD,), jnp.float32),
    mesh=_mesh,
    scratch_types=[
        pltpu.VMEM_SHARED((NPAD,), jnp.float32),
        pltpu.VMEM_SHARED((NPAD,), jnp.float32),
        pltpu.VMEM((BE,), jnp.int32),
        pltpu.VMEM((BE,), jnp.int32),
        pltpu.VMEM((BE,), jnp.int32),
        pltpu.VMEM((BE,), jnp.int32),
        pltpu.VMEM((BE,), jnp.float32),
        pltpu.VMEM((PER_SUB,), jnp.float32),
        pltpu.SemaphoreType.DMA,
        pltpu.SemaphoreType.DMA,
    ],
)


def _p2_body(src_hbm, dst_hbm, v2c0_hbm, v2c1_hbm, zeros_hbm, out0_hbm, out1_hbm,
             v0_sh, w1_sh, a0_sh, a1_sh, idx_sa, idx_da, idx_sb, idx_db,
             vals0, vals1, stage_v, sem_a, sem_b):
    c = lax.axis_index("c")
    s = lax.axis_index("s")
    wid = c * SUB + s
    sl = pl.ds(s * PER_SUB, PER_SUB)
    base_e = wid * EPW
    _start_idx(src_hbm, base_e, idx_sa, sem_a)
    _start_idx(dst_hbm, base_e, idx_da, sem_a)
    pltpu.sync_copy(v2c0_hbm.at[sl], stage_v)
    pltpu.sync_copy(stage_v, v0_sh.at[sl])
    pltpu.sync_copy(v2c1_hbm.at[sl], stage_v)
    pltpu.sync_copy(stage_v, w1_sh.at[sl])
    pltpu.sync_copy(zeros_hbm.at[sl], stage_v)
    pltpu.sync_copy(stage_v, a0_sh.at[sl])
    pltpu.sync_copy(stage_v, a1_sh.at[sl])
    plsc.subcore_barrier()

    def do_batch(idx_s, idx_d):
        pltpu.sync_copy(v0_sh.at[idx_s], vals0)
        pltpu.sync_copy(w1_sh.at[idx_s], vals1)
        pltpu.sync_copy(vals0, a0_sh.at[idx_d], add=True)
        pltpu.sync_copy(vals1, a1_sh.at[idx_d], add=True)

    def outer(i, carry):
        e0 = base_e + i * (2 * BE)
        _start_idx(src_hbm, e0 + BE, idx_sb, sem_b)
        _start_idx(dst_hbm, e0 + BE, idx_db, sem_b)
        _drain_idx(src_hbm, idx_sa, sem_a)
        _drain_idx(dst_hbm, idx_da, sem_a)
        do_batch(idx_sa, idx_da)
        _start_idx(src_hbm, e0 + 2 * BE, idx_sa, sem_a)
        _start_idx(dst_hbm, e0 + 2 * BE, idx_da, sem_a)
        _drain_idx(src_hbm, idx_sb, sem_b)
        _drain_idx(dst_hbm, idx_db, sem_b)
        do_batch(idx_sb, idx_db)
        return carry

    lax.fori_loop(0, TB // 2, outer, 0)
    _drain_idx(src_hbm, idx_sa, sem_a)
    _drain_idx(dst_hbm, idx_da, sem_a)
    plsc.subcore_barrier()
    pltpu.sync_copy(a0_sh.at[sl], stage_v)
    pltpu.sync_copy(stage_v, out0_hbm.at[pl.ds(c * NPAD + s * PER_SUB, PER_SUB)])
    pltpu.sync_copy(a1_sh.at[sl], stage_v)
    pltpu.sync_copy(stage_v, out1_hbm.at[pl.ds(c * NPAD + s * PER_SUB, PER_SUB)])


_p2_call = pl.kernel(
    _p2_body,
    out_type=(
        jax.ShapeDtypeStruct((CORES * NPAD,), jnp.float32),
        jax.ShapeDtypeStruct((CORES * NPAD,), jnp.float32),
    ),
    mesh=_mesh,
    scratch_types=[
        pltpu.VMEM_SHARED((NPAD,), jnp.float32),
        pltpu.VMEM_SHARED((NPAD,), jnp.float32),
        pltpu.VMEM_SHARED((NPAD,), jnp.float32),
        pltpu.VMEM_SHARED((NPAD,), jnp.float32),
        pltpu.VMEM((BE,), jnp.int32),
        pltpu.VMEM((BE,), jnp.int32),
        pltpu.VMEM((BE,), jnp.int32),
        pltpu.VMEM((BE,), jnp.int32),
        pltpu.VMEM((BE,), jnp.float32),
        pltpu.VMEM((BE,), jnp.float32),
        pltpu.VMEM((PER_SUB,), jnp.float32),
        pltpu.SemaphoreType.DMA,
        pltpu.SemaphoreType.DMA,
    ],
)


def _prep_body(degp_ref, xp_ref, dinv_ref, v1_ref):
    deg = degp_ref[0] + degp_ref[1] + 1.0
    dinv = lax.rsqrt(deg)
    dinv_ref[...] = dinv
    v1_ref[...] = xp_ref[...] * dinv


_prep_call = pl.pallas_call(
    _prep_body,
    out_shape=(
        jax.ShapeDtypeStruct((NROWS, 128), jnp.float32),
        jax.ShapeDtypeStruct((NROWS, 128), jnp.float32),
    ),
)


def _mid_body(accp_ref, dinv_ref, v1_ref, w1_ref, b1_ref, w2_ref, v2cm_ref):
    dinv = dinv_ref[...]
    s1 = dinv * (accp_ref[0] + accp_ref[1] + v1_ref[...])
    g0 = jnp.zeros_like(s1)
    g1 = jnp.zeros_like(s1)
    for k in range(16):
        h = jnp.maximum(s1 * w1_ref[0, k] + b1_ref[0, k], 0.0)
        g0 += h * w2_ref[k, 0]
        g1 += h * w2_ref[k, 1]
    v2cm_ref[0] = g0 * dinv
    v2cm_ref[1] = g1 * dinv


_mid_call = pl.pallas_call(
    _mid_body,
    in_specs=[
        pl.BlockSpec(memory_space=pltpu.VMEM),
        pl.BlockSpec(memory_space=pltpu.VMEM),
        pl.BlockSpec(memory_space=pltpu.VMEM),
        pl.BlockSpec(memory_space=pltpu.SMEM),
        pl.BlockSpec(memory_space=pltpu.SMEM),
        pl.BlockSpec(memory_space=pltpu.SMEM),
    ],
    out_shape=jax.ShapeDtypeStruct((2, NROWS, 128), jnp.float32),
)


def _final_body(acc2cm_ref, v2cm_ref, dinv_ref, b2_ref, batch_ref, out_ref):
    dinv = dinv_ref[...]
    hs = []
    for ch in range(2):
        s2 = dinv * (acc2cm_ref[0, ch] + acc2cm_ref[1, ch] + v2cm_ref[ch])
        hs.append(jnp.maximum(s2 + b2_ref[0, ch], 0.0))
    bt = batch_ref[...]
    for g in range(NG):
        m = (bt == g).astype(jnp.float32)
        cnt = jnp.maximum(jnp.sum(m), 1.0)
        z0 = jnp.sum(m * hs[0]) / cnt
        z1 = jnp.sum(m * hs[1]) / cnt
        mx = jnp.maximum(z0, z1)
        lse = jnp.log(jnp.exp(z0 - mx) + jnp.exp(z1 - mx)) + mx
        out_ref[g, 0] = z0 - lse
        out_ref[g, 1] = z1 - lse


_final_call = pl.pallas_call(
    _final_body,
    in_specs=[
        pl.BlockSpec(memory_space=pltpu.VMEM),
        pl.BlockSpec(memory_space=pltpu.VMEM),
        pl.BlockSpec(memory_space=pltpu.VMEM),
        pl.BlockSpec(memory_space=pltpu.SMEM),
        pl.BlockSpec(memory_space=pltpu.VMEM),
    ],
    out_specs=pl.BlockSpec(memory_space=pltpu.SMEM),
    out_shape=jax.ShapeDtypeStruct((NG, 2), jnp.float32),
)


def kernel(x, edge_index, batch, W1, b1, W2, b2):
    ei = edge_index.astype(jnp.int32)
    npad_e = EP_ALLOC - NE
    pad_idx = NN + (jnp.arange(npad_e, dtype=jnp.int32) % (NPAD - NN))
    srcp = jnp.concatenate([ei[0], pad_idx])
    dstp = jnp.concatenate([ei[1], pad_idx])
    zerosN = jnp.zeros((NPAD,), jnp.float32)
    ones_b = jnp.ones((BE,), jnp.float32)

    degp = _deg_call(dstp, ones_b, zerosN)

    xp = jnp.concatenate([x[:, 0], jnp.zeros((NPAD - NN,), jnp.float32)])
    dinv, v1 = _prep_call(degp.reshape(CORES, NROWS, 128), xp.reshape(NROWS, 128))

    acc1p = _p1_call(srcp, dstp, v1.reshape(NPAD), zerosN)

    v2cm = _mid_call(acc1p.reshape(CORES, NROWS, 128), dinv, v1,
                     W1, b1.reshape(1, 16), W2)

    v2flat = v2cm.reshape(2, NPAD)
    acc2c0, acc2c1 = _p2_call(srcp, dstp, v2flat[0], v2flat[1], zerosN)

    acc2cm = jnp.stack(
        [acc2c0.reshape(CORES, NROWS, 128), acc2c1.reshape(CORES, NROWS, 128)],
        axis=1)
    batchp = jnp.concatenate(
        [batch.astype(jnp.int32), jnp.full((NPAD - NN,), NG, jnp.int32)]
    ).reshape(NROWS, 128)

    return _final_call(acc2cm, v2cm, dinv, b2.reshape(1, 2), batchp)


# speedup vs baseline: 1.7998x; 1.0330x over previous
import jax
import jax.numpy as jnp
from jax import lax
from jax.experimental import pallas as pl
from jax.experimental.pallas import tpu as pltpu
from jax.experimental.pallas import tpu_sc as plsc

NN = 100000
NE = 3200000
NG = 64
NPAD = 100096
NROWS = NPAD // 128
SUB = 16
CORES = 2
NW = CORES * SUB
PER_SUB = NPAD // SUB

EPW = NE // NW
BE = 10000
TB = EPW // BE

_mesh = plsc.VectorSubcoreMesh(core_axis_name="c", subcore_axis_name="s")


def _start_idx(edge_hbm, e0, buf, sem):
    return pltpu.async_copy(edge_hbm.at[pl.ds(e0, BE)], buf, sem)


def _drain_idx(edge_hbm, buf, sem):
    pltpu.make_async_copy(edge_hbm.at[pl.ds(0, BE)], buf, sem).wait()


def _deg_body(dst_hbm, ones_hbm, zeros_hbm, out_hbm,
              deg_sh, idx_a, idx_b, ones_v, stage_v, sem_a, sem_b):
    c = lax.axis_index("c")
    s = lax.axis_index("s")
    wid = c * SUB + s
    sl = pl.ds(s * PER_SUB, PER_SUB)
    base_e = wid * EPW
    _start_idx(dst_hbm, base_e, idx_a, sem_a)
    pltpu.sync_copy(zeros_hbm.at[sl], stage_v)
    pltpu.sync_copy(stage_v, deg_sh.at[sl])
    pltpu.sync_copy(ones_hbm, ones_v)
    plsc.subcore_barrier()

    def outer(i, carry):
        e0 = base_e + i * (2 * BE)
        _start_idx(dst_hbm, e0 + BE, idx_b, sem_b)
        _drain_idx(dst_hbm, idx_a, sem_a)
        pltpu.sync_copy(ones_v, deg_sh.at[idx_a], add=True)
        _start_idx(dst_hbm, e0 + 2 * BE, idx_a, sem_a)
        _drain_idx(dst_hbm, idx_b, sem_b)
        pltpu.sync_copy(ones_v, deg_sh.at[idx_b], add=True)
        return carry

    lax.fori_loop(0, TB // 2 - 1, outer, 0)
    _start_idx(dst_hbm, base_e + (TB - 1) * BE, idx_b, sem_b)
    _drain_idx(dst_hbm, idx_a, sem_a)
    pltpu.sync_copy(ones_v, deg_sh.at[idx_a], add=True)
    _drain_idx(dst_hbm, idx_b, sem_b)
    pltpu.sync_copy(ones_v, deg_sh.at[idx_b], add=True)
    plsc.subcore_barrier()
    pltpu.sync_copy(deg_sh.at[sl], stage_v)
    pltpu.sync_copy(stage_v, out_hbm.at[pl.ds(c * NPAD + s * PER_SUB, PER_SUB)])


_deg_call = pl.kernel(
    _deg_body,
    out_type=jax.ShapeDtypeStruct((CORES * NPAD,), jnp.float32),
    mesh=_mesh,
    scratch_types=[
        pltpu.VMEM_SHARED((NPAD,), jnp.float32),
        pltpu.VMEM((BE,), jnp.int32),
        pltpu.VMEM((BE,), jnp.int32),
        pltpu.VMEM((BE,), jnp.float32),
        pltpu.VMEM((PER_SUB,), jnp.float32),
        pltpu.SemaphoreType.DMA,
        pltpu.SemaphoreType.DMA,
    ],
)


def _p1_body(src_hbm, dst_hbm, v1_hbm, zeros_hbm, out_hbm,
             v1_sh, acc_sh, idx_sa, idx_da, idx_sb, idx_db, vals, stage_v,
             sem_a, sem_b):
    c = lax.axis_index("c")
    s = lax.axis_index("s")
    wid = c * SUB + s
    sl = pl.ds(s * PER_SUB, PER_SUB)
    base_e = wid * EPW
    _start_idx(src_hbm, base_e, idx_sa, sem_a)
    _start_idx(dst_hbm, base_e, idx_da, sem_a)
    pltpu.sync_copy(v1_hbm.at[sl], stage_v)
    pltpu.sync_copy(stage_v, v1_sh.at[sl])
    pltpu.sync_copy(zeros_hbm.at[sl], stage_v)
    pltpu.sync_copy(stage_v, acc_sh.at[sl])
    plsc.subcore_barrier()

    def outer(i, carry):
        e0 = base_e + i * (2 * BE)
        _start_idx(src_hbm, e0 + BE, idx_sb, sem_b)
        _start_idx(dst_hbm, e0 + BE, idx_db, sem_b)
        _drain_idx(src_hbm, idx_sa, sem_a)
        _drain_idx(dst_hbm, idx_da, sem_a)
        pltpu.sync_copy(v1_sh.at[idx_sa], vals)
        pltpu.sync_copy(vals, acc_sh.at[idx_da], add=True)
        _start_idx(src_hbm, e0 + 2 * BE, idx_sa, sem_a)
        _start_idx(dst_hbm, e0 + 2 * BE, idx_da, sem_a)
        _drain_idx(src_hbm, idx_sb, sem_b)
        _drain_idx(dst_hbm, idx_db, sem_b)
        pltpu.sync_copy(v1_sh.at[idx_sb], vals)
        pltpu.sync_copy(vals, acc_sh.at[idx_db], add=True)
        return carry

    lax.fori_loop(0, TB // 2 - 1, outer, 0)
    e9 = base_e + (TB - 1) * BE
    _start_idx(src_hbm, e9, idx_sb, sem_b)
    _start_idx(dst_hbm, e9, idx_db, sem_b)
    _drain_idx(src_hbm, idx_sa, sem_a)
    _drain_idx(dst_hbm, idx_da, sem_a)
    pltpu.sync_copy(v1_sh.at[idx_sa], vals)
    pltpu.sync_copy(vals, acc_sh.at[idx_da], add=True)
    _drain_idx(src_hbm, idx_sb, sem_b)
    _drain_idx(dst_hbm, idx_db, sem_b)
    pltpu.sync_copy(v1_sh.at[idx_sb], vals)
    pltpu.sync_copy(vals, acc_sh.at[idx_db], add=True)
    plsc.subcore_barrier()
    pltpu.sync_copy(acc_sh.at[sl], stage_v)
    pltpu.sync_copy(stage_v, out_hbm.at[pl.ds(c * NPAD + s * PER_SUB, PER_SUB)])


_p1_call = pl.kernel(
    _p1_body,
    out_type=jax.ShapeDtypeStruct((CORES * NPAD,), jnp.float32),
    mesh=_mesh,
    scratch_types=[
        pltpu.VMEM_SHARED((NPAD,), jnp.float32),
        pltpu.VMEM_SHARED((NPAD,), jnp.float32),
        pltpu.VMEM((BE,), jnp.int32),
        pltpu.VMEM((BE,), jnp.int32),
        pltpu.VMEM((BE,), jnp.int32),
        pltpu.VMEM((BE,), jnp.int32),
        pltpu.VMEM((BE,), jnp.float32),
        pltpu.VMEM((PER_SUB,), jnp.float32),
        pltpu.SemaphoreType.DMA,
        pltpu.SemaphoreType.DMA,
    ],
)


def _p2_body(src_hbm, dst_hbm, v2c0_hbm, v2c1_hbm, zeros_hbm, out0_hbm, out1_hbm,
             v0_sh, w1_sh, a0_sh, a1_sh, idx_sa, idx_da, idx_sb, idx_db,
             vals0, vals1, stage_v, sem_a, sem_b):
    c = lax.axis_index("c")
    s = lax.axis_index("s")
    wid = c * SUB + s
    sl = pl.ds(s * PER_SUB, PER_SUB)
    base_e = wid * EPW
    _start_idx(src_hbm, base_e, idx_sa, sem_a)
    _start_idx(dst_hbm, base_e, idx_da, sem_a)
    pltpu.sync_copy(v2c0_hbm.at[sl], stage_v)
    pltpu.sync_copy(stage_v, v0_sh.at[sl])
    pltpu.sync_copy(v2c1_hbm.at[sl], stage_v)
    pltpu.sync_copy(stage_v, w1_sh.at[sl])
    pltpu.sync_copy(zeros_hbm.at[sl], stage_v)
    pltpu.sync_copy(stage_v, a0_sh.at[sl])
    pltpu.sync_copy(stage_v, a1_sh.at[sl])
    plsc.subcore_barrier()

    def do_batch(idx_s, idx_d):
        pltpu.sync_copy(v0_sh.at[idx_s], vals0)
        pltpu.sync_copy(w1_sh.at[idx_s], vals1)
        pltpu.sync_copy(vals0, a0_sh.at[idx_d], add=True)
        pltpu.sync_copy(vals1, a1_sh.at[idx_d], add=True)

    def outer(i, carry):
        e0 = base_e + i * (2 * BE)
        _start_idx(src_hbm, e0 + BE, idx_sb, sem_b)
        _start_idx(dst_hbm, e0 + BE, idx_db, sem_b)
        _drain_idx(src_hbm, idx_sa, sem_a)
        _drain_idx(dst_hbm, idx_da, sem_a)
        do_batch(idx_sa, idx_da)
        _start_idx(src_hbm, e0 + 2 * BE, idx_sa, sem_a)
        _start_idx(dst_hbm, e0 + 2 * BE, idx_da, sem_a)
        _drain_idx(src_hbm, idx_sb, sem_b)
        _drain_idx(dst_hbm, idx_db, sem_b)
        do_batch(idx_sb, idx_db)
        return carry

    lax.fori_loop(0, TB // 2 - 1, outer, 0)
    e9 = base_e + (TB - 1) * BE
    _start_idx(src_hbm, e9, idx_sb, sem_b)
    _start_idx(dst_hbm, e9, idx_db, sem_b)
    _drain_idx(src_hbm, idx_sa, sem_a)
    _drain_idx(dst_hbm, idx_da, sem_a)
    do_batch(idx_sa, idx_da)
    _drain_idx(src_hbm, idx_sb, sem_b)
    _drain_idx(dst_hbm, idx_db, sem_b)
    do_batch(idx_sb, idx_db)
    plsc.subcore_barrier()
    pltpu.sync_copy(a0_sh.at[sl], stage_v)
    pltpu.sync_copy(stage_v, out0_hbm.at[pl.ds(c * NPAD + s * PER_SUB, PER_SUB)])
    pltpu.sync_copy(a1_sh.at[sl], stage_v)
    pltpu.sync_copy(stage_v, out1_hbm.at[pl.ds(c * NPAD + s * PER_SUB, PER_SUB)])


_p2_call = pl.kernel(
    _p2_body,
    out_type=(
        jax.ShapeDtypeStruct((CORES * NPAD,), jnp.float32),
        jax.ShapeDtypeStruct((CORES * NPAD,), jnp.float32),
    ),
    mesh=_mesh,
    scratch_types=[
        pltpu.VMEM_SHARED((NPAD,), jnp.float32),
        pltpu.VMEM_SHARED((NPAD,), jnp.float32),
        pltpu.VMEM_SHARED((NPAD,), jnp.float32),
        pltpu.VMEM_SHARED((NPAD,), jnp.float32),
        pltpu.VMEM((BE,), jnp.int32),
        pltpu.VMEM((BE,), jnp.int32),
        pltpu.VMEM((BE,), jnp.int32),
        pltpu.VMEM((BE,), jnp.int32),
        pltpu.VMEM((BE,), jnp.float32),
        pltpu.VMEM((BE,), jnp.float32),
        pltpu.VMEM((PER_SUB,), jnp.float32),
        pltpu.SemaphoreType.DMA,
        pltpu.SemaphoreType.DMA,
    ],
)


def _prep_body(degp_ref, xp_ref, dinv_ref, v1_ref):
    deg = degp_ref[0] + degp_ref[1] + 1.0
    dinv = lax.rsqrt(deg)
    dinv_ref[...] = dinv
    v1_ref[...] = xp_ref[...] * dinv


_prep_call = pl.pallas_call(
    _prep_body,
    out_shape=(
        jax.ShapeDtypeStruct((NROWS, 128), jnp.float32),
        jax.ShapeDtypeStruct((NROWS, 128), jnp.float32),
    ),
)


def _mid_body(accp_ref, dinv_ref, v1_ref, w1_ref, b1_ref, w2_ref, v2cm_ref):
    dinv = dinv_ref[...]
    s1 = dinv * (accp_ref[0] + accp_ref[1] + v1_ref[...])
    g0 = jnp.zeros_like(s1)
    g1 = jnp.zeros_like(s1)
    for k in range(16):
        h = jnp.maximum(s1 * w1_ref[0, k] + b1_ref[0, k], 0.0)
        g0 += h * w2_ref[k, 0]
        g1 += h * w2_ref[k, 1]
    v2cm_ref[0] = g0 * dinv
    v2cm_ref[1] = g1 * dinv


_mid_call = pl.pallas_call(
    _mid_body,
    in_specs=[
        pl.BlockSpec(memory_space=pltpu.VMEM),
        pl.BlockSpec(memory_space=pltpu.VMEM),
        pl.BlockSpec(memory_space=pltpu.VMEM),
        pl.BlockSpec(memory_space=pltpu.SMEM),
        pl.BlockSpec(memory_space=pltpu.SMEM),
        pl.BlockSpec(memory_space=pltpu.SMEM),
    ],
    out_shape=jax.ShapeDtypeStruct((2, NROWS, 128), jnp.float32),
)


def _final_body(acc2cm_ref, v2cm_ref, dinv_ref, b2_ref, batch_ref, out_ref):
    dinv = dinv_ref[...]
    hs = []
    for ch in range(2):
        s2 = dinv * (acc2cm_ref[0, ch] + acc2cm_ref[1, ch] + v2cm_ref[ch])
        hs.append(jnp.maximum(s2 + b2_ref[0, ch], 0.0))
    bt = batch_ref[...]
    for g in range(NG):
        m = (bt == g).astype(jnp.float32)
        cnt = jnp.maximum(jnp.sum(m), 1.0)
        z0 = jnp.sum(m * hs[0]) / cnt
        z1 = jnp.sum(m * hs[1]) / cnt
        mx = jnp.maximum(z0, z1)
        lse = jnp.log(jnp.exp(z0 - mx) + jnp.exp(z1 - mx)) + mx
        out_ref[g, 0] = z0 - lse
        out_ref[g, 1] = z1 - lse


_final_call = pl.pallas_call(
    _final_body,
    in_specs=[
        pl.BlockSpec(memory_space=pltpu.VMEM),
        pl.BlockSpec(memory_space=pltpu.VMEM),
        pl.BlockSpec(memory_space=pltpu.VMEM),
        pl.BlockSpec(memory_space=pltpu.SMEM),
        pl.BlockSpec(memory_space=pltpu.VMEM),
    ],
    out_specs=pl.BlockSpec(memory_space=pltpu.SMEM),
    out_shape=jax.ShapeDtypeStruct((NG, 2), jnp.float32),
)


def kernel(x, edge_index, batch, W1, b1, W2, b2):
    ei = edge_index.astype(jnp.int32)
    srcp = ei[0]
    dstp = ei[1]
    zerosN = jnp.zeros((NPAD,), jnp.float32)
    ones_b = jnp.ones((BE,), jnp.float32)

    degp = _deg_call(dstp, ones_b, zerosN)

    xp = jnp.concatenate([x[:, 0], jnp.zeros((NPAD - NN,), jnp.float32)])
    dinv, v1 = _prep_call(degp.reshape(CORES, NROWS, 128), xp.reshape(NROWS, 128))

    acc1p = _p1_call(srcp, dstp, v1.reshape(NPAD), zerosN)

    v2cm = _mid_call(acc1p.reshape(CORES, NROWS, 128), dinv, v1,
                     W1, b1.reshape(1, 16), W2)

    v2flat = v2cm.reshape(2, NPAD)
    acc2c0, acc2c1 = _p2_call(srcp, dstp, v2flat[0], v2flat[1], zerosN)

    acc2cm = jnp.stack(
        [acc2c0.reshape(CORES, NROWS, 128), acc2c1.reshape(CORES, NROWS, 128)],
        axis=1)
    batchp = jnp.concatenate(
        [batch.astype(jnp.int32), jnp.full((NPAD - NN,), NG, jnp.int32)]
    ).reshape(NROWS, 128)

    return _final_call(acc2cm, v2cm, dinv, b2.reshape(1, 2), batchp)
